# baseline (speedup 1.0000x reference)
import functools

import jax
import jax.numpy as jnp
from jax import lax
from jax.experimental import pallas as pl
from jax.experimental.pallas import tpu as pltpu
from jax.experimental.pallas import tpu_sc as plsc

_NUM_CORES = 2
_NUM_SUBCORES = 16
_NSHIFT = 16
_PLANE = 4096
_BLK = 16


@functools.partial(jax.jit, static_argnums=(1, 2))
def _expand_bias(rev16b, H, S):
    NW = _NUM_CORES * _NUM_SUBCORES
    NBLK = H * (S // _BLK)
    BPW = NBLK // NW
    ABLK = S // _BLK

    mesh = plsc.VectorSubcoreMesh(core_axis_name="c", subcore_axis_name="s")

    @functools.partial(
        pl.kernel,
        out_type=jax.ShapeDtypeStruct((H, S, S), jnp.float32),
        mesh=mesh,
        scratch_types=[
            pltpu.VMEM((2, _BLK, S), jnp.float32),
            pltpu.SemaphoreType.DMA,
            pltpu.SemaphoreType.DMA,
        ],
        compiler_params=pltpu.CompilerParams(use_tc_tiling_on_sc=False),
    )
    def body(rev_hbm, out_hbm, buf, gsem, ssem):
        wid = lax.axis_index("s") * _NUM_CORES + lax.axis_index("c")
        h = wid // 2
        a0 = (wid % 2) * BPW

        def start_gather(t, slot):
            q = (ABLK - 1) - (a0 + t)
            pltpu.make_async_copy(
                rev_hbm.at[:, h, pl.ds(q * _NSHIFT, S)], buf.at[slot], gsem
            ).start()

        def start_scatter(t, slot):
            pltpu.make_async_copy(
                buf.at[slot],
                out_hbm.at[h, pl.ds((a0 + t) * _BLK, _BLK), :],
                ssem,
            ).start()

        def wait_gather_one():
            pltpu.make_async_copy(
                rev_hbm.at[:, 0, pl.ds(0, S)], buf.at[0], gsem
            ).wait()

        def wait_scatter_one():
            pltpu.make_async_copy(
                out_hbm.at[0, pl.ds(0, _BLK), :],
                out_hbm.at[0, pl.ds(0, _BLK), :],
                ssem,
            ).wait()

        start_gather(0, 0)

        def step(t, carry):
            slot = t % 2
            wait_gather_one()
            start_scatter(t, slot)

            @pl.when(t + 1 < BPW)
            def _next():
                @pl.when(t >= 1)
                def _free():
                    wait_scatter_one()
                start_gather(t + 1, 1 - slot)

            return carry

        lax.fori_loop(0, BPW, step, 0)
        pltpu.make_async_copy(
            out_hbm.at[0, pl.ds(0, 2 * _BLK), :],
            out_hbm.at[0, pl.ds(0, 2 * _BLK), :],
            ssem,
        ).wait()

    return body(rev16b)


def kernel(seq_len, table):
    del seq_len
    R, H = table.shape
    S = (R + 1) // 2
    rev = table[::-1, :].T
    rev_pad = jnp.pad(rev, ((0, 0), (0, _PLANE + _NSHIFT - 1 - rev.shape[1])))
    rev16b = jnp.stack(
        [rev_pad[:, s:s + _PLANE] for s in reversed(range(_NSHIFT))]
    )
    rows = _expand_bias(rev16b, H, S)
    return rows.reshape(1, H, S, S)


# speedup vs baseline: 1.2496x; 1.2496x over previous
import functools

import jax
import jax.numpy as jnp
from jax.experimental import pallas as pl
from jax.experimental.pallas import tpu as pltpu

_NSHIFT = 128
_PLANE = 4096
_BLK = 128


@functools.partial(jax.jit, static_argnums=(1, 2))
def _expand_bias_tc(rev128c, H, S):
    ABLK = S // _BLK

    def body(in_ref, out_ref):
        a = pl.program_id(1)
        q = (ABLK - 1) - a
        off = pl.multiple_of(q * _BLK, 128)
        out_ref[0] = in_ref[0, :, pl.ds(off, S)]

    return pl.pallas_call(
        body,
        grid=(H, ABLK),
        in_specs=[pl.BlockSpec((1, _NSHIFT, _PLANE), lambda h, a: (h, 0, 0))],
        out_specs=pl.BlockSpec((1, _BLK, S), lambda h, a: (h, a, 0)),
        out_shape=jax.ShapeDtypeStruct((H, S, S), jnp.float32),
        compiler_params=pltpu.CompilerParams(
            dimension_semantics=("parallel", "arbitrary"),
        ),
    )(rev128c)


def kernel(seq_len, table):
    del seq_len
    R, H = table.shape
    S = (R + 1) // 2
    rev = table[::-1, :].T
    rev_pad = jnp.pad(rev, ((0, 0), (0, _PLANE + _NSHIFT - 1 - rev.shape[1])))
    rev128c = jnp.stack(
        [rev_pad[:, s:s + _PLANE] for s in reversed(range(_NSHIFT))], axis=1
    )
    rows = _expand_bias_tc(rev128c, H, S)
    return rows.reshape(1, H, S, S)
